# baseline (speedup 1.0000x reference)
import functools

import jax
import jax.numpy as jnp
from jax import lax
from jax.experimental import pallas as pl
from jax.experimental.pallas import tpu as pltpu
from jax.experimental.pallas import tpu_sc as plsc

_N_EDGE_ROWS = 1600001
_PAD_ROWS = 1601536
_TOTAL = 4096 * 4 * 16
_LANES = 128
_ROWS = _TOTAL // _LANES
_NW = 32
_CPW = _ROWS // _NW
_BLK = 8
_GBLK = 16


def _sc_gather_one(paths2d, tab):
    mesh = plsc.VectorSubcoreMesh(core_axis_name="c", subcore_axis_name="s")
    nblk = _CPW // _GBLK

    @functools.partial(
        pl.kernel,
        out_type=jax.ShapeDtypeStruct((_ROWS, _LANES), jnp.float32),
        mesh=mesh,
        scratch_types=[
            pltpu.VMEM((_CPW, _LANES), jnp.int32),
            pltpu.VMEM((_CPW, _LANES), jnp.float32),
            pltpu.SemaphoreType.DMA,
        ],
    )
    def k(paths_hbm, tab_hbm, out_hbm, idx_v, val_v, sem):
        wid = lax.axis_index("s") * 2 + lax.axis_index("c")
        row0 = wid * _CPW
        pltpu.sync_copy(paths_hbm.at[pl.ds(row0, _CPW)], idx_v)

        def fire(b):
            for j in range(_GBLK):
                c = b * _GBLK + j
                pltpu.async_copy(tab_hbm.at[idx_v.at[c]], val_v.at[c], sem)

        def drain(b):
            for j in range(_GBLK):
                c = b * _GBLK + j
                pltpu.make_async_copy(
                    tab_hbm.at[idx_v.at[c]], val_v.at[c], sem).wait()

        fire(0)
        fire(1)

        def body(b, carry):
            fire(b + 2)
            drain(b)
            return carry

        lax.fori_loop(0, nblk - 2, body, 0)
        drain(nblk - 2)
        drain(nblk - 1)
        pltpu.sync_copy(val_v, out_hbm.at[pl.ds(row0, _CPW)])

    return k(paths2d, tab)


def _tc_softplus_reduce(vals, paths2d, default_distance):
    def body(v_ref, p_ref, dd_ref, td_ref, fnd_ref):
        v = v_ref[...]
        mf = (p_ref[...] != 0).astype(jnp.float32)
        sp = jnp.maximum(v, 0.0) + jnp.log(1.0 + jnp.exp(-jnp.abs(v)))
        sp5 = sp.reshape(4, 2, 32, 8, _LANES)
        mf5 = mf.reshape(4, 2, 32, 8, _LANES)
        td = jnp.zeros((4, 32, _LANES), jnp.float32)
        cnt = jnp.zeros((4, 32, _LANES), jnp.float32)
        for jt in range(2):
            for ji in range(8):
                td = td + sp5[:, jt, :, ji, :]
                cnt = cnt + mf5[:, jt, :, ji, :]
        fnd = cnt > 0.0
        td_ref[...] = jnp.where(fnd, td, dd_ref[0, 0])
        fnd_ref[...] = fnd.astype(jnp.int32)

    return pl.pallas_call(
        body,
        out_shape=[
            jax.ShapeDtypeStruct((4, 32, _LANES), jnp.float32),
            jax.ShapeDtypeStruct((4, 32, _LANES), jnp.int32),
        ],
        in_specs=[
            pl.BlockSpec(memory_space=pltpu.VMEM),
            pl.BlockSpec(memory_space=pltpu.VMEM),
            pl.BlockSpec(memory_space=pltpu.SMEM),
        ],
    )(vals, paths2d, default_distance)


def _tc_logsig_reduce(vals):
    def body(v_ref, lp_ref):
        v = v_ref[...]
        ls = jnp.minimum(v, 0.0) - jnp.log(1.0 + jnp.exp(-jnp.abs(v)))
        ls5 = ls.reshape(4, 2, 32, 8, _LANES)
        lp = jnp.zeros((4, 32, _LANES), jnp.float32)
        for jt in range(2):
            for ji in range(8):
                lp = lp + ls5[:, jt, :, ji, :]
        lp_ref[...] = lp

    return pl.pallas_call(
        body,
        out_shape=jax.ShapeDtypeStruct((4, 32, _LANES), jnp.float32),
        in_specs=[pl.BlockSpec(memory_space=pltpu.VMEM)],
    )(vals)


def _flatten_table(tab2d, dummy_val):
    ext = jnp.concatenate(
        [tab2d, jnp.zeros((_PAD_ROWS - _N_EDGE_ROWS, 1), jnp.float32)],
        axis=0)
    ri = lax.broadcasted_iota(jnp.int32, (_PAD_ROWS, 1), 0)
    ext = jnp.where(ri == 0, dummy_val, ext)
    return ext.reshape(_PAD_ROWS)


def kernel(from_ix, to_ix, target_paths, edge_weight_logits,
           edge_adjacency_logits, default_distance):
    paths2d = (target_paths.reshape(32, 128, 4, 2, 8)
               .transpose(2, 3, 0, 4, 1)
               .reshape(_ROWS, _LANES))
    w_tab = _flatten_table(edge_weight_logits, -100.0)
    w_vals = _sc_gather_one(paths2d, w_tab)
    a_tab = _flatten_table(edge_adjacency_logits, 100.0)
    a_vals = _sc_gather_one(paths2d, a_tab)
    td, fnd = _tc_softplus_reduce(w_vals, paths2d, default_distance)
    lp = _tc_logsig_reduce(a_vals)
    shape = target_paths.shape[:-1]
    td = td.transpose(1, 2, 0).reshape(shape)
    lp = lp.transpose(1, 2, 0).reshape(shape)
    fnd = fnd.transpose(1, 2, 0).reshape(shape)
    return td, lp, fnd.astype(jnp.bool_)


# speedup vs baseline: 1.0502x; 1.0502x over previous
import functools

import jax
import jax.numpy as jnp
from jax import lax
from jax.experimental import pallas as pl
from jax.experimental.pallas import tpu as pltpu
from jax.experimental.pallas import tpu_sc as plsc

_N_EDGE_ROWS = 1600001
_PAD_ROWS = 1601536
_TOTAL = 4096 * 4 * 16
_LANES = 128
_ROWS = _TOTAL // _LANES
_NW = 32
_CPW = _ROWS // _NW
_BLK = 8
_GBLK = 16


def _sc_gather_one(paths2d, tab):
    mesh = plsc.VectorSubcoreMesh(core_axis_name="c", subcore_axis_name="s")
    nblk = _CPW // _GBLK

    @functools.partial(
        pl.kernel,
        out_type=jax.ShapeDtypeStruct((_ROWS, _LANES), jnp.float32),
        mesh=mesh,
        scratch_types=[
            pltpu.VMEM((_CPW, _LANES), jnp.int32),
            pltpu.VMEM((_CPW, _LANES), jnp.float32),
            pltpu.SemaphoreType.DMA,
        ],
    )
    def k(paths_hbm, tab_hbm, out_hbm, idx_v, val_v, sem):
        wid = lax.axis_index("s") * 2 + lax.axis_index("c")
        row0 = wid * _CPW
        pltpu.sync_copy(paths_hbm.at[pl.ds(row0, _CPW)], idx_v)

        def fire(b):
            for j in range(_GBLK):
                c = b * _GBLK + j
                pltpu.async_copy(tab_hbm.at[idx_v.at[c]], val_v.at[c], sem)

        def drain(b):
            for j in range(_GBLK):
                c = b * _GBLK + j
                pltpu.make_async_copy(
                    tab_hbm.at[idx_v.at[c]], val_v.at[c], sem).wait()

        fire(0)
        fire(1)
        fire(2)

        def body(b, carry):
            fire(b + 3)
            drain(b)
            return carry

        lax.fori_loop(0, nblk - 3, body, 0)
        drain(nblk - 3)
        drain(nblk - 2)
        drain(nblk - 1)
        pltpu.sync_copy(val_v, out_hbm.at[pl.ds(row0, _CPW)])

    return k(paths2d, tab)


def _tc_softplus_reduce(vals, paths2d, default_distance, last_v):
    def body(v_ref, p_ref, dd_ref, lv_ref, td_ref, fnd_ref):
        p = p_ref[...]
        v = jnp.where(p == (_N_EDGE_ROWS - 1), lv_ref[0, 0], v_ref[...])
        mf = (p != 0).astype(jnp.float32)
        sp = (jnp.maximum(v, 0.0) + jnp.log(1.0 + jnp.exp(-jnp.abs(v)))) * mf
        sp5 = sp.reshape(4, 2, 32, 8, _LANES)
        mf5 = mf.reshape(4, 2, 32, 8, _LANES)
        td = jnp.zeros((4, 32, _LANES), jnp.float32)
        cnt = jnp.zeros((4, 32, _LANES), jnp.float32)
        for jt in range(2):
            for ji in range(8):
                td = td + sp5[:, jt, :, ji, :]
                cnt = cnt + mf5[:, jt, :, ji, :]
        fnd = cnt > 0.0
        td_ref[...] = jnp.where(fnd, td, dd_ref[0, 0])
        fnd_ref[...] = fnd.astype(jnp.int32)

    return pl.pallas_call(
        body,
        out_shape=[
            jax.ShapeDtypeStruct((4, 32, _LANES), jnp.float32),
            jax.ShapeDtypeStruct((4, 32, _LANES), jnp.int32),
        ],
        in_specs=[
            pl.BlockSpec(memory_space=pltpu.VMEM),
            pl.BlockSpec(memory_space=pltpu.VMEM),
            pl.BlockSpec(memory_space=pltpu.SMEM),
            pl.BlockSpec(memory_space=pltpu.SMEM),
        ],
    )(vals, paths2d, default_distance, last_v)


def _tc_logsig_reduce(vals, paths2d, last_v):
    def body(v_ref, p_ref, lv_ref, lp_ref):
        p = p_ref[...]
        v = jnp.where(p == (_N_EDGE_ROWS - 1), lv_ref[0, 0], v_ref[...])
        mf = (p != 0).astype(jnp.float32)
        ls = (jnp.minimum(v, 0.0) - jnp.log(1.0 + jnp.exp(-jnp.abs(v)))) * mf
        ls5 = ls.reshape(4, 2, 32, 8, _LANES)
        lp = jnp.zeros((4, 32, _LANES), jnp.float32)
        for jt in range(2):
            for ji in range(8):
                lp = lp + ls5[:, jt, :, ji, :]
        lp_ref[...] = lp

    return pl.pallas_call(
        body,
        out_shape=jax.ShapeDtypeStruct((4, 32, _LANES), jnp.float32),
        in_specs=[
            pl.BlockSpec(memory_space=pltpu.VMEM),
            pl.BlockSpec(memory_space=pltpu.VMEM),
            pl.BlockSpec(memory_space=pltpu.SMEM),
        ],
    )(vals, paths2d, last_v)


def _flatten_table(tab2d):
    ext = jnp.concatenate(
        [tab2d, jnp.zeros((_PAD_ROWS - _N_EDGE_ROWS, 1), jnp.float32)],
        axis=0)
    return ext.reshape(_PAD_ROWS)


def kernel(from_ix, to_ix, target_paths, edge_weight_logits,
           edge_adjacency_logits, default_distance):
    paths2d = (target_paths.reshape(32, 128, 4, 2, 8)
               .transpose(2, 3, 0, 4, 1)
               .reshape(_ROWS, _LANES))
    w_tab = _flatten_table(edge_weight_logits)
    w_vals = _sc_gather_one(paths2d, w_tab)
    a_tab = _flatten_table(edge_adjacency_logits)
    a_vals = _sc_gather_one(paths2d, a_tab)
    last_w = edge_weight_logits[_N_EDGE_ROWS - 1:, :]
    last_a = edge_adjacency_logits[_N_EDGE_ROWS - 1:, :]
    td, fnd = _tc_softplus_reduce(w_vals, paths2d, default_distance, last_w)
    lp = _tc_logsig_reduce(a_vals, paths2d, last_a)
    shape = target_paths.shape[:-1]
    td = td.transpose(1, 2, 0).reshape(shape)
    lp = lp.transpose(1, 2, 0).reshape(shape)
    fnd = fnd.transpose(1, 2, 0).reshape(shape)
    return td, lp, fnd.astype(jnp.bool_)


# speedup vs baseline: 1.0787x; 1.0272x over previous
import functools

import jax
import jax.numpy as jnp
from jax import lax
from jax.experimental import pallas as pl
from jax.experimental.pallas import tpu as pltpu
from jax.experimental.pallas import tpu_sc as plsc

_N_EDGE_ROWS = 1600001
_PAD_ROWS = 1601536
_TOTAL = 4096 * 4 * 16
_LANES = 128
_ROWS = _TOTAL // _LANES
_NW = 32
_CPW = _ROWS // _NW
_BLK = 8
_GBLK = 16


def _sc_gather_one(paths2d, tab):
    mesh = plsc.VectorSubcoreMesh(core_axis_name="c", subcore_axis_name="s")
    nblk = _CPW // _GBLK

    @functools.partial(
        pl.kernel,
        out_type=jax.ShapeDtypeStruct((_ROWS, _LANES), jnp.float32),
        mesh=mesh,
        scratch_types=[
            pltpu.VMEM((_CPW, _LANES), jnp.int32),
            pltpu.VMEM((_CPW, _LANES), jnp.float32),
            pltpu.SemaphoreType.DMA,
        ],
    )
    def k(paths_hbm, tab_hbm, out_hbm, idx_v, val_v, sem):
        wid = lax.axis_index("s") * 2 + lax.axis_index("c")
        row0 = wid * _CPW
        pltpu.sync_copy(paths_hbm.at[pl.ds(row0, _CPW)], idx_v)

        def fire(b):
            for j in range(_GBLK):
                c = b * _GBLK + j
                pltpu.async_copy(tab_hbm.at[idx_v.at[c]], val_v.at[c], sem)

        def drain(b):
            for j in range(_GBLK):
                c = b * _GBLK + j
                pltpu.make_async_copy(
                    tab_hbm.at[idx_v.at[c]], val_v.at[c], sem).wait()

        fire(0)
        fire(1)
        fire(2)

        def body(b, carry):
            fire(b + 3)
            drain(b)
            return carry

        lax.fori_loop(0, nblk - 3, body, 0)
        drain(nblk - 3)
        drain(nblk - 2)
        drain(nblk - 1)
        pltpu.sync_copy(val_v, out_hbm.at[pl.ds(row0, _CPW)])

    return k(paths2d, tab)


def _tc_softplus_reduce(vals, paths2d, default_distance):
    def body(v_ref, p_ref, dd_ref, td_ref, fnd_ref):
        p = p_ref[...]
        v = v_ref[...]
        mf = (p != 0).astype(jnp.float32)
        sp = (jnp.maximum(v, 0.0) + jnp.log(1.0 + jnp.exp(-jnp.abs(v)))) * mf
        sp5 = sp.reshape(4, 2, 32, 8, _LANES)
        mf5 = mf.reshape(4, 2, 32, 8, _LANES)
        td = jnp.zeros((4, 32, _LANES), jnp.float32)
        cnt = jnp.zeros((4, 32, _LANES), jnp.float32)
        for jt in range(2):
            for ji in range(8):
                td = td + sp5[:, jt, :, ji, :]
                cnt = cnt + mf5[:, jt, :, ji, :]
        fnd = cnt > 0.0
        td_ref[...] = jnp.where(fnd, td, dd_ref[0, 0])
        fnd_ref[...] = fnd.astype(jnp.int32)

    return pl.pallas_call(
        body,
        out_shape=[
            jax.ShapeDtypeStruct((4, 32, _LANES), jnp.float32),
            jax.ShapeDtypeStruct((4, 32, _LANES), jnp.int32),
        ],
        in_specs=[
            pl.BlockSpec(memory_space=pltpu.VMEM),
            pl.BlockSpec(memory_space=pltpu.VMEM),
            pl.BlockSpec(memory_space=pltpu.SMEM),
        ],
    )(vals, paths2d, default_distance)


def _tc_logsig_reduce(vals):
    def body(v_ref, lp_ref):
        v = v_ref[...]
        ls = jnp.minimum(v, 0.0) - jnp.log(1.0 + jnp.exp(-jnp.abs(v)))
        ls5 = ls.reshape(4, 2, 32, 8, _LANES)
        lp = jnp.zeros((4, 32, _LANES), jnp.float32)
        for jt in range(2):
            for ji in range(8):
                lp = lp + ls5[:, jt, :, ji, :]
        lp_ref[...] = lp

    return pl.pallas_call(
        body,
        out_shape=jax.ShapeDtypeStruct((4, 32, _LANES), jnp.float32),
        in_specs=[pl.BlockSpec(memory_space=pltpu.VMEM)],
    )(vals)


def _flatten_table(tab2d, dummy_val=None):
    ext = jnp.concatenate(
        [tab2d, jnp.zeros((_PAD_ROWS - _N_EDGE_ROWS, 1), jnp.float32)],
        axis=0)
    if dummy_val is not None:
        ri = lax.broadcasted_iota(jnp.int32, (_PAD_ROWS, 1), 0)
        ext = jnp.where(ri == 0, dummy_val, ext)
    return ext.reshape(_PAD_ROWS)


def kernel(from_ix, to_ix, target_paths, edge_weight_logits,
           edge_adjacency_logits, default_distance):
    paths2d = (target_paths.reshape(32, 128, 4, 2, 8)
               .transpose(2, 3, 0, 4, 1)
               .reshape(_ROWS, _LANES))
    w_tab = _flatten_table(edge_weight_logits)
    w_vals = _sc_gather_one(paths2d, w_tab)
    a_tab = _flatten_table(edge_adjacency_logits, 100.0)
    a_vals = _sc_gather_one(paths2d, a_tab)
    td, fnd = _tc_softplus_reduce(w_vals, paths2d, default_distance)
    lp = _tc_logsig_reduce(a_vals)
    shape = target_paths.shape[:-1]
    td = td.transpose(1, 2, 0).reshape(shape)
    lp = lp.transpose(1, 2, 0).reshape(shape)
    fnd = fnd.transpose(1, 2, 0).reshape(shape)
    return td, lp, fnd.astype(jnp.bool_)


# speedup vs baseline: 1.0927x; 1.0130x over previous
import functools

import jax
import jax.numpy as jnp
from jax import lax
from jax.experimental import pallas as pl
from jax.experimental.pallas import tpu as pltpu
from jax.experimental.pallas import tpu_sc as plsc

_N_EDGE_ROWS = 1600001
_PAD_ROWS = 1601536
_TOTAL = 4096 * 4 * 16
_LANES = 128
_ROWS = _TOTAL // _LANES
_NW = 32
_CPW = _ROWS // _NW
_BLK = 8
_GBLK = 16


def _sc_gather_one(paths2d, tab):
    mesh = plsc.VectorSubcoreMesh(core_axis_name="c", subcore_axis_name="s")
    nblk = _CPW // _GBLK

    @functools.partial(
        pl.kernel,
        out_type=jax.ShapeDtypeStruct((_ROWS, _LANES), jnp.float32),
        mesh=mesh,
        scratch_types=[
            pltpu.VMEM((_CPW, _LANES), jnp.int32),
            pltpu.VMEM((_CPW, _LANES), jnp.float32),
            pltpu.SemaphoreType.DMA,
        ],
    )
    def k(paths_hbm, tab_hbm, out_hbm, idx_v, val_v, sem):
        wid = lax.axis_index("s") * 2 + lax.axis_index("c")
        row0 = wid * _CPW
        pltpu.sync_copy(paths_hbm.at[pl.ds(row0, _CPW)], idx_v)

        def fire(b):
            for j in range(_GBLK):
                c = b * _GBLK + j
                pltpu.async_copy(tab_hbm.at[idx_v.at[c]], val_v.at[c], sem)

        def drain(b):
            for j in range(_GBLK):
                c = b * _GBLK + j
                pltpu.make_async_copy(
                    tab_hbm.at[idx_v.at[c]], val_v.at[c], sem).wait()

        def body(b, carry):
            fire(b)
            return carry

        lax.fori_loop(0, nblk, body, 0)

        def body2(b, carry):
            drain(b)
            return carry

        lax.fori_loop(0, nblk, body2, 0)
        pltpu.sync_copy(val_v, out_hbm.at[pl.ds(row0, _CPW)])

    return k(paths2d, tab)


def _tc_softplus_reduce(vals, paths2d, default_distance):
    def body(v_ref, p_ref, dd_ref, td_ref, fnd_ref):
        p = p_ref[...]
        v = v_ref[...]
        mf = (p != 0).astype(jnp.float32)
        sp = (jnp.maximum(v, 0.0) + jnp.log(1.0 + jnp.exp(-jnp.abs(v)))) * mf
        sp5 = sp.reshape(4, 2, 32, 8, _LANES)
        mf5 = mf.reshape(4, 2, 32, 8, _LANES)
        td = jnp.zeros((4, 32, _LANES), jnp.float32)
        cnt = jnp.zeros((4, 32, _LANES), jnp.float32)
        for jt in range(2):
            for ji in range(8):
                td = td + sp5[:, jt, :, ji, :]
                cnt = cnt + mf5[:, jt, :, ji, :]
        fnd = cnt > 0.0
        td_ref[...] = jnp.where(fnd, td, dd_ref[0, 0])
        fnd_ref[...] = fnd.astype(jnp.int32)

    return pl.pallas_call(
        body,
        out_shape=[
            jax.ShapeDtypeStruct((4, 32, _LANES), jnp.float32),
            jax.ShapeDtypeStruct((4, 32, _LANES), jnp.int32),
        ],
        in_specs=[
            pl.BlockSpec(memory_space=pltpu.VMEM),
            pl.BlockSpec(memory_space=pltpu.VMEM),
            pl.BlockSpec(memory_space=pltpu.SMEM),
        ],
    )(vals, paths2d, default_distance)


def _tc_logsig_reduce(vals):
    def body(v_ref, lp_ref):
        v = v_ref[...]
        ls = jnp.minimum(v, 0.0) - jnp.log(1.0 + jnp.exp(-jnp.abs(v)))
        ls5 = ls.reshape(4, 2, 32, 8, _LANES)
        lp = jnp.zeros((4, 32, _LANES), jnp.float32)
        for jt in range(2):
            for ji in range(8):
                lp = lp + ls5[:, jt, :, ji, :]
        lp_ref[...] = lp

    return pl.pallas_call(
        body,
        out_shape=jax.ShapeDtypeStruct((4, 32, _LANES), jnp.float32),
        in_specs=[pl.BlockSpec(memory_space=pltpu.VMEM)],
    )(vals)


def _flatten_table(tab2d, dummy_val=None):
    ext = jnp.concatenate(
        [tab2d, jnp.zeros((_PAD_ROWS - _N_EDGE_ROWS, 1), jnp.float32)],
        axis=0)
    if dummy_val is not None:
        ri = lax.broadcasted_iota(jnp.int32, (_PAD_ROWS, 1), 0)
        ext = jnp.where(ri == 0, dummy_val, ext)
    return ext.reshape(_PAD_ROWS)


def kernel(from_ix, to_ix, target_paths, edge_weight_logits,
           edge_adjacency_logits, default_distance):
    paths2d = (target_paths.reshape(32, 128, 4, 2, 8)
               .transpose(2, 3, 0, 4, 1)
               .reshape(_ROWS, _LANES))
    w_tab = _flatten_table(edge_weight_logits)
    w_vals = _sc_gather_one(paths2d, w_tab)
    a_tab = _flatten_table(edge_adjacency_logits, 100.0)
    a_vals = _sc_gather_one(paths2d, a_tab)
    td, fnd = _tc_softplus_reduce(w_vals, paths2d, default_distance)
    lp = _tc_logsig_reduce(a_vals)
    shape = target_paths.shape[:-1]
    td = td.transpose(1, 2, 0).reshape(shape)
    lp = lp.transpose(1, 2, 0).reshape(shape)
    fnd = fnd.transpose(1, 2, 0).reshape(shape)
    return td, lp, fnd.astype(jnp.bool_)
